# baseline (speedup 1.0000x reference)
import jax
import jax.numpy as jnp
from jax.experimental import pallas as pl
from jax.experimental.pallas import tpu as pltpu

_BM = 200
_NBUF = 4


def _fused_kernel(a_hbm, x_ref, w_ref, b_ref, o_ref, bufs, z_ref, y_ref, sems):
    n = x_ref.shape[0]
    nblk = n // _BM
    nsteps = 2 * nblk

    z_ref[...] = jnp.dot(
        x_ref[...].astype(jnp.bfloat16),
        w_ref[...].astype(jnp.bfloat16),
        preferred_element_type=jnp.float32,
    ).astype(jnp.bfloat16)

    def _copy(step, slot):
        blk = jax.lax.rem(step, nblk)
        return pltpu.make_async_copy(
            a_hbm.at[pl.ds(blk * _BM, _BM), :],
            bufs.at[slot],
            sems.at[slot],
        )

    for s in range(_NBUF):
        _copy(jnp.int32(s), jnp.int32(s)).start()

    def loop_body(step, _):
        slot = jax.lax.rem(step, _NBUF)
        blk = jax.lax.rem(step, nblk)
        _copy(step, slot).wait()
        a_bf = bufs[slot].astype(jnp.bfloat16)

        @pl.when(step < nblk)
        def _():
            y_ref[pl.ds(blk * _BM, _BM), :] = jnp.dot(
                a_bf, z_ref[...], preferred_element_type=jnp.float32
            ).astype(jnp.bfloat16)

        @pl.when(step >= nblk)
        def _():
            acc = jnp.dot(a_bf, y_ref[...], preferred_element_type=jnp.float32)
            v = acc + b_ref[...]
            m = jnp.max(v, axis=1, keepdims=True)
            lse = jnp.log(jnp.sum(jnp.exp(v - m), axis=1, keepdims=True)) + m
            o_ref[pl.ds(blk * _BM, _BM), :] = v - lse

        @pl.when(step + _NBUF < nsteps)
        def _():
            _copy(step + _NBUF, slot).start()

        return 0

    jax.lax.fori_loop(0, nsteps, loop_body, 0)


def kernel(x, adjs, weight, bias):
    n, d_in = x.shape
    d_out = weight.shape[1]
    a = adjs.reshape(n, n)
    bias2d = bias.reshape(1, d_out)

    return pl.pallas_call(
        _fused_kernel,
        in_specs=[
            pl.BlockSpec(memory_space=pl.ANY),
            pl.BlockSpec(memory_space=pltpu.VMEM),
            pl.BlockSpec(memory_space=pltpu.VMEM),
            pl.BlockSpec(memory_space=pltpu.VMEM),
        ],
        out_specs=pl.BlockSpec(memory_space=pltpu.VMEM),
        out_shape=jax.ShapeDtypeStruct((n, d_out), jnp.float32),
        scratch_shapes=[
            pltpu.VMEM((_NBUF, _BM, n), jnp.float32),
            pltpu.VMEM((n, d_out), jnp.bfloat16),
            pltpu.VMEM((n, d_out), jnp.bfloat16),
            pltpu.SemaphoreType.DMA((_NBUF,)),
        ],
    )(a, x, weight, bias2d)


# speedup vs baseline: 1.0910x; 1.0910x over previous
import jax
import jax.numpy as jnp
from jax.experimental import pallas as pl
from jax.experimental.pallas import tpu as pltpu

_BM = 200
_NB32 = 3
_NBO = 2
_NB8 = 2
_F8 = jnp.float8_e4m3fn


def _fused_kernel(a_hbm, x_ref, w_ref, b_ref, o_ref, a8_hbm,
                  bufs32, bufs8o, bufs8i, z_ref, y32_ref, ybf_ref,
                  sem32, semo, sem8):
    n = x_ref.shape[0]
    nblk = n // _BM
    nsteps = 2 * nblk

    z_ref[...] = jnp.dot(
        x_ref[...].astype(jnp.bfloat16),
        w_ref[...].astype(jnp.bfloat16),
        preferred_element_type=jnp.float32,
    ).astype(jnp.bfloat16)

    def _copy32(blk):
        return pltpu.make_async_copy(
            a_hbm.at[pl.ds(blk * _BM, _BM), :],
            bufs32.at[jax.lax.rem(blk, _NB32)],
            sem32.at[jax.lax.rem(blk, _NB32)],
        )

    def _copy8in(blk):
        return pltpu.make_async_copy(
            a8_hbm.at[blk],
            bufs8i.at[jax.lax.rem(blk, _NB8)],
            sem8.at[jax.lax.rem(blk, _NB8)],
        )

    def _copy8out(blk):
        return pltpu.make_async_copy(
            bufs8o.at[jax.lax.rem(blk, _NBO)],
            a8_hbm.at[blk],
            semo.at[jax.lax.rem(blk, _NBO)],
        )

    for s in range(_NB32):
        _copy32(jnp.int32(s)).start()

    def loop_body(step, _):
        @pl.when(step < nblk)
        def _():
            blk = step
            _copy32(blk).wait()
            a32 = bufs32[jax.lax.rem(blk, _NB32)]
            y32_ref[pl.ds(blk * _BM, _BM), :] = jnp.dot(
                a32.astype(jnp.bfloat16), z_ref[...],
                preferred_element_type=jnp.float32,
            )

            @pl.when(blk >= _NBO)
            def _():
                _copy8out(blk - _NBO).wait()

            bufs8o[jax.lax.rem(blk, _NBO)] = a32.astype(_F8)
            _copy8out(blk).start()

        @pl.when(step >= nblk)
        def _():
            blk = step - nblk

            @pl.when(blk == 0)
            def _():
                for k in range(_NBO):
                    _copy8out(nblk - _NBO + k).wait()
                ybf_ref[...] = y32_ref[...].astype(jnp.bfloat16)

            _copy8in(blk).wait()
            a_bf = bufs8i[jax.lax.rem(blk, _NB8)].astype(jnp.bfloat16)
            acc = jnp.dot(a_bf, ybf_ref[...], preferred_element_type=jnp.float32)
            v = acc + b_ref[...]
            m = jnp.max(v, axis=1, keepdims=True)
            lse = jnp.log(jnp.sum(jnp.exp(v - m), axis=1, keepdims=True)) + m
            o_ref[pl.ds(blk * _BM, _BM), :] = v - lse

        t32 = step + _NB32
        @pl.when(t32 < nblk)
        def _():
            _copy32(t32).start()

        t8 = step + _NB8
        @pl.when(jnp.logical_and(t8 >= nblk, t8 < nsteps))
        def _():
            _copy8in(t8 - nblk).start()

        return 0

    jax.lax.fori_loop(0, nsteps, loop_body, 0)


def kernel(x, adjs, weight, bias):
    n, d_in = x.shape
    d_out = weight.shape[1]
    a = adjs.reshape(n, n)
    bias2d = bias.reshape(1, d_out)
    nblk = n // _BM

    out, _ = pl.pallas_call(
        _fused_kernel,
        in_specs=[
            pl.BlockSpec(memory_space=pl.ANY),
            pl.BlockSpec(memory_space=pltpu.VMEM),
            pl.BlockSpec(memory_space=pltpu.VMEM),
            pl.BlockSpec(memory_space=pltpu.VMEM),
        ],
        out_specs=[
            pl.BlockSpec(memory_space=pltpu.VMEM),
            pl.BlockSpec(memory_space=pl.ANY),
        ],
        out_shape=[
            jax.ShapeDtypeStruct((n, d_out), jnp.float32),
            jax.ShapeDtypeStruct((nblk, _BM, n), _F8),
        ],
        scratch_shapes=[
            pltpu.VMEM((_NB32, _BM, n), jnp.float32),
            pltpu.VMEM((_NBO, _BM, n), _F8),
            pltpu.VMEM((_NB8, _BM, n), _F8),
            pltpu.VMEM((n, d_out), jnp.bfloat16),
            pltpu.VMEM((n, d_out), jnp.float32),
            pltpu.VMEM((n, d_out), jnp.bfloat16),
            pltpu.SemaphoreType.DMA((_NB32,)),
            pltpu.SemaphoreType.DMA((_NBO,)),
            pltpu.SemaphoreType.DMA((_NB8,)),
        ],
    )(a, x, weight, bias2d)
    return out
